# Initial kernel scaffold; baseline (speedup 1.0000x reference)
import jax
import jax.numpy as jnp
from jax.experimental import pallas as pl


def kernel(feature, item_embedding, W_ih, W_hh, b_ih, b_hh, h0):
    raise NotImplementedError("write your pallas kernel here")


# speedup vs baseline: 3.9976x; 3.9976x over previous
import jax
import jax.numpy as jnp
from jax.experimental import pallas as pl
from jax.experimental.pallas import tpu as pltpu


def _rnn_update_kernel(feature_smem, idxs_vmem, w_cat, bias, h0_ref,
                       table_in, table_out, buf, sem):
    L = idxs_vmem.shape[0]

    def gather_start(i, _):
        idx = feature_smem[i, 0]
        pltpu.make_async_copy(table_out.at[pl.ds(idx, 1), :],
                              buf.at[pl.ds(i, 1), :], sem).start()
        return 0

    jax.lax.fori_loop(0, L, gather_start, 0)

    def gather_wait(i, _):
        idx = feature_smem[i, 0]
        pltpu.make_async_copy(table_out.at[pl.ds(idx, 1), :],
                              buf.at[pl.ds(i, 1), :], sem).wait()
        return 0

    jax.lax.fori_loop(0, L, gather_wait, 0)

    wc = w_cat[...]
    b = bias[...]
    idxs = idxs_vmem[...]

    def step(i, h):
        x = buf[pl.ds(i, 1), :]
        xh = jnp.concatenate([x, h], axis=1)
        h_new = jnp.tanh(
            jnp.dot(xh, wc, preferred_element_type=jnp.float32) + b)
        t_i = feature_smem[i, 1]
        prev = jnp.where(i == 0, L - 1, i - 1)
        dt = (t_i - feature_smem[prev, 1]).astype(jnp.float32)
        idx_i = feature_smem[i, 0]
        buf[...] = jnp.where(idxs == idx_i, h_new, buf[...])
        return h_new * (1.0 / dt + 1.0)

    jax.lax.fori_loop(0, L, step, h0_ref[...])

    def scatter_start(i, _):
        idx = feature_smem[i, 0]
        pltpu.make_async_copy(buf.at[pl.ds(i, 1), :],
                              table_out.at[pl.ds(idx, 1), :], sem).start()
        return 0

    jax.lax.fori_loop(0, L, scatter_start, 0)

    def scatter_wait(i, _):
        idx = feature_smem[i, 0]
        pltpu.make_async_copy(buf.at[pl.ds(i, 1), :],
                              table_out.at[pl.ds(idx, 1), :], sem).wait()
        return 0

    jax.lax.fori_loop(0, L, scatter_wait, 0)


def kernel(feature, item_embedding, W_ih, W_hh, b_ih, b_hh, h0):
    L = feature.shape[0]
    M, D = item_embedding.shape
    H = W_ih.shape[0]
    w_cat = jnp.concatenate([W_ih, W_hh], axis=1).T
    bias = (b_ih + b_hh).reshape(1, H)
    idxs2d = feature[:, 0:1]
    h02d = h0.reshape(1, H)

    return pl.pallas_call(
        _rnn_update_kernel,
        out_shape=jax.ShapeDtypeStruct((M, D), item_embedding.dtype),
        in_specs=[
            pl.BlockSpec(memory_space=pltpu.MemorySpace.SMEM),
            pl.BlockSpec(memory_space=pltpu.MemorySpace.VMEM),
            pl.BlockSpec(memory_space=pltpu.MemorySpace.VMEM),
            pl.BlockSpec(memory_space=pltpu.MemorySpace.VMEM),
            pl.BlockSpec(memory_space=pltpu.MemorySpace.VMEM),
            pl.BlockSpec(memory_space=pltpu.MemorySpace.HBM),
        ],
        out_specs=pl.BlockSpec(memory_space=pltpu.MemorySpace.HBM),
        input_output_aliases={5: 0},
        scratch_shapes=[
            pltpu.VMEM((L, D), jnp.float32),
            pltpu.SemaphoreType.DMA,
        ],
    )(feature, idxs2d, w_cat, bias, h02d, item_embedding)
